# Initial kernel scaffold; baseline (speedup 1.0000x reference)
import jax
import jax.numpy as jnp
from jax.experimental import pallas as pl


def kernel(inputs, embed_table, W, b):
    raise NotImplementedError("write your pallas kernel here")


# speedup vs baseline: 1.0881x; 1.0881x over previous
import functools

import jax
import jax.numpy as jnp
from jax import lax
from jax.experimental import pallas as pl
from jax.experimental.pallas import tpu as pltpu
from jax.experimental.pallas import tpu_sc as plsc

_NC = 2
_NS = 16
_NW = _NC * _NS
_R = 512
_N = 4 * _R
_J = 4


def _fuse_body(t_ref, w_ref, b_ref, o_ref):
    o_ref[...] = (
        jnp.dot(t_ref[...], w_ref[...], preferred_element_type=jnp.float32)
        + b_ref[...]
    )


def _fuse_table(table, W, b):
    V, E = table.shape
    O = W.shape[0]
    rows_per = 64
    w_in = rows_per * E
    w_out = rows_per * O
    flat_rows = V // rows_per
    t_flat = table.reshape(flat_rows, w_in)
    w_big = jnp.kron(jnp.eye(rows_per, dtype=jnp.float32), W.T)
    b_tile = jnp.tile(b, rows_per)[None, :]
    blk = 1024
    grid = (flat_rows + blk - 1) // blk
    out_flat = pl.pallas_call(
        _fuse_body,
        grid=(grid,),
        in_specs=[
            pl.BlockSpec((blk, w_in), lambda i: (i, 0)),
            pl.BlockSpec((w_in, w_out), lambda i: (0, 0)),
            pl.BlockSpec((1, w_out), lambda i: (0, 0)),
        ],
        out_specs=pl.BlockSpec((blk, w_out), lambda i: (i, 0)),
        out_shape=jax.ShapeDtypeStruct((flat_rows, w_out), jnp.float32),
    )(t_flat, w_big, b_tile)
    return out_flat.reshape(V, O)


def _sc_gather(fused, idx2d):
    n_rows = idx2d.shape[0]
    O = fused.shape[1]
    rows_per_w = n_rows // _NW
    stages = rows_per_w // _J
    mesh = plsc.VectorSubcoreMesh(
        core_axis_name="c", subcore_axis_name="s",
        num_cores=_NC, num_subcores=_NS,
    )

    @functools.partial(
        pl.kernel,
        mesh=mesh,
        compiler_params=pltpu.CompilerParams(use_tc_tiling_on_sc=False),
        out_type=jax.ShapeDtypeStruct((n_rows, _R, O), jnp.float32),
        scratch_types=[
            pltpu.VMEM((_J, _N), jnp.int32),
            pltpu.VMEM((_J, _N, O), jnp.float32),
            pltpu.SemaphoreType.DMA,
        ],
    )
    def gather_kernel(table_hbm, idx_hbm, out_hbm, idx_v, rows_v, sem):
        wid = lax.axis_index("s") * _NC + lax.axis_index("c")
        base_row = wid * rows_per_w

        def stage(s, carry):
            r0 = base_row + s * _J
            pltpu.sync_copy(idx_hbm.at[pl.ds(r0, _J)], idx_v)
            copies = [
                pltpu.async_copy(
                    table_hbm.at[idx_v.at[j]], rows_v.at[j], sem
                )
                for j in range(_J)
            ]
            for j, c in enumerate(copies):
                c.wait()
                pltpu.sync_copy(
                    rows_v.at[j, pl.ds(0, _R)], out_hbm.at[r0 + j]
                )
            return carry

        lax.fori_loop(0, stages, stage, 0)

    return gather_kernel(fused, idx2d)


def kernel(inputs, embed_table, W, b):
    B, L = inputs.shape
    O = W.shape[0]
    fused = _fuse_table(embed_table, W, b)
    idx2d = inputs.reshape(-1, _R)
    spread = jnp.zeros(idx2d.shape + (4,), jnp.int32)
    spread = spread.at[:, :, 0].set(idx2d * 4).reshape(idx2d.shape[0], _N)
    out3 = _sc_gather(fused, spread)
    return out3.reshape(B, L, O)


# speedup vs baseline: 10.1063x; 9.2883x over previous
import functools

import jax
import jax.numpy as jnp
from jax import lax
from jax.experimental import pallas as pl
from jax.experimental.pallas import tpu as pltpu
from jax.experimental.pallas import tpu_sc as plsc

_NC = 2
_NS = 16
_NW = _NC * _NS
_D = 128
_K = 512
_O = 2


def _fuse_body(t_ref, w_ref, b_ref, o_ref):
    o_ref[...] = (
        jnp.dot(t_ref[...], w_ref[...], preferred_element_type=jnp.float32)
        + b_ref[...]
    )


def _fuse_table(table, W, b):
    V, E = table.shape
    O = W.shape[0]
    rows_per = 64
    w_in = rows_per * E
    w_out = rows_per * _D
    flat_rows = V // rows_per
    t_flat = table.reshape(flat_rows, w_in)
    w_pad = jnp.zeros((_D, E), jnp.float32).at[:O].set(W)
    b_pad = jnp.zeros((_D,), jnp.float32).at[:O].set(b)
    w_big = jnp.kron(jnp.eye(rows_per, dtype=jnp.float32), w_pad.T)
    b_tile = jnp.tile(b_pad, rows_per)[None, :]
    blk = 256
    grid = (flat_rows + blk - 1) // blk
    out_flat = pl.pallas_call(
        _fuse_body,
        grid=(grid,),
        in_specs=[
            pl.BlockSpec((blk, w_in), lambda i: (i, 0)),
            pl.BlockSpec((w_in, w_out), lambda i: (0, 0)),
            pl.BlockSpec((1, w_out), lambda i: (0, 0)),
        ],
        out_specs=pl.BlockSpec((blk, w_out), lambda i: (i, 0)),
        out_shape=jax.ShapeDtypeStruct((flat_rows, w_out), jnp.float32),
    )(t_flat, w_big, b_tile)
    return out_flat.reshape(V, _D)


def _sc_gather(fused, idx):
    n = idx.shape[0]
    per_w = n // _NW
    stages = per_w // _K
    mesh = plsc.VectorSubcoreMesh(
        core_axis_name="c", subcore_axis_name="s",
        num_cores=_NC, num_subcores=_NS,
    )

    @functools.partial(
        pl.kernel,
        mesh=mesh,
        compiler_params=pltpu.CompilerParams(needs_layout_passes=False),
        out_type=jax.ShapeDtypeStruct((n * _O,), jnp.float32),
        scratch_types=[
            pltpu.VMEM((_K,), jnp.int32),
            pltpu.VMEM((_K, _D), jnp.float32),
            pltpu.VMEM((_K * _O,), jnp.float32),
            pltpu.SemaphoreType.DMA,
        ],
    )
    def gather_kernel(table_hbm, idx_hbm, out_hbm, idx_v, rows_v, com_v, sem):
        wid = lax.axis_index("s") * _NC + lax.axis_index("c")
        base = wid * per_w
        lane = lax.iota(jnp.int32, 16)
        live = lane < _O

        def stage(s, carry):
            p0 = base + s * _K
            pltpu.sync_copy(idx_hbm.at[pl.ds(p0, _K)], idx_v)
            pltpu.async_copy(table_hbm.at[idx_v], rows_v, sem).wait()

            def compact(g, c2):
                for u in range(8):
                    r = g * 8 + u
                    v = rows_v[r, pl.ds(0, 16)]
                    plsc.store_scatter(
                        com_v, [lane + r * _O], v, mask=live
                    )
                return c2

            lax.fori_loop(0, _K // 8, compact, 0)
            pltpu.sync_copy(com_v, out_hbm.at[pl.ds(p0 * _O, _K * _O)])
            return carry

        lax.fori_loop(0, stages, stage, 0)

    return gather_kernel(fused, idx)


def kernel(inputs, embed_table, W, b):
    B, L = inputs.shape
    O = W.shape[0]
    fused = _fuse_table(embed_table, W, b)
    out2 = _sc_gather(fused, inputs.reshape(-1))
    return out2.reshape(B, L, O)


# speedup vs baseline: 16.6638x; 1.6489x over previous
import functools

import jax
import jax.numpy as jnp
from jax import lax
from jax.experimental import pallas as pl
from jax.experimental.pallas import tpu as pltpu
from jax.experimental.pallas import tpu_sc as plsc

_NC = 2
_NS = 16
_NW = _NC * _NS
_D = 128
_K = 512
_O = 2
_PB = 16384


def _fuse_body(t_ref, w_ref, b_ref, o_ref):
    o_ref[...] = (
        jnp.dot(t_ref[...], w_ref[...], preferred_element_type=jnp.float32)
        + b_ref[...]
    )


def _fuse_table(table, W, b):
    V, E = table.shape
    O = W.shape[0]
    rows_per = 64
    w_in = rows_per * E
    w_out = rows_per * _D
    flat_rows = V // rows_per
    t_flat = table.reshape(flat_rows, w_in)
    w_pad = jnp.zeros((_D, E), jnp.float32).at[:O].set(W)
    b_pad = jnp.zeros((_D,), jnp.float32).at[:O].set(b)
    w_big = jnp.kron(jnp.eye(rows_per, dtype=jnp.float32), w_pad.T)
    b_tile = jnp.tile(b_pad, rows_per)[None, :]
    blk = 256
    grid = (flat_rows + blk - 1) // blk
    out_flat = pl.pallas_call(
        _fuse_body,
        grid=(grid,),
        in_specs=[
            pl.BlockSpec((blk, w_in), lambda i: (i, 0)),
            pl.BlockSpec((w_in, w_out), lambda i: (0, 0)),
            pl.BlockSpec((1, w_out), lambda i: (0, 0)),
        ],
        out_specs=pl.BlockSpec((blk, w_out), lambda i: (i, 0)),
        out_shape=jax.ShapeDtypeStruct((flat_rows, w_out), jnp.float32),
    )(t_flat, w_big, b_tile)
    return out_flat.reshape(V, _D)


def _sc_gather(fused, idx):
    n = idx.shape[0]
    per_w = n // _NW
    stages = per_w // _K
    mesh = plsc.VectorSubcoreMesh(
        core_axis_name="c", subcore_axis_name="s",
        num_cores=_NC, num_subcores=_NS,
    )

    @functools.partial(
        pl.kernel,
        mesh=mesh,
        compiler_params=pltpu.CompilerParams(needs_layout_passes=False),
        out_type=jax.ShapeDtypeStruct((n * _O,), jnp.float32),
        scratch_types=[
            pltpu.VMEM((_K,), jnp.int32),
            pltpu.VMEM((_K, _D), jnp.float32),
            pltpu.VMEM((_K * _O,), jnp.float32),
            pltpu.SemaphoreType.DMA,
        ],
    )
    def gather_kernel(table_hbm, idx_hbm, out_hbm, idx_v, rows_v, com_v, sem):
        wid = lax.axis_index("s") * _NC + lax.axis_index("c")
        base = wid * per_w
        lane = lax.iota(jnp.int32, 16)
        live = lane < _O

        def stage(s, carry):
            p0 = base + s * _K
            h = p0 // _PB
            b0 = p0 - h * _PB
            o_base = h * (_O * _PB) + b0
            pltpu.sync_copy(idx_hbm.at[pl.ds(p0, _K)], idx_v)
            pltpu.async_copy(table_hbm.at[idx_v], rows_v, sem).wait()

            def compact(g, c2):
                for u in range(8):
                    r = g * 8 + u
                    v = rows_v[r, pl.ds(0, 16)]
                    plsc.store_scatter(
                        com_v, [lane * _K + r], v, mask=live
                    )
                return c2

            lax.fori_loop(0, _K // 8, compact, 0)
            pltpu.sync_copy(
                com_v.at[pl.ds(0, _K)], out_hbm.at[pl.ds(o_base, _K)]
            )
            pltpu.sync_copy(
                com_v.at[pl.ds(_K, _K)],
                out_hbm.at[pl.ds(o_base + _PB, _K)],
            )
            return carry

        lax.fori_loop(0, stages, stage, 0)

    return gather_kernel(fused, idx)


def kernel(inputs, embed_table, W, b):
    B, L = inputs.shape
    O = W.shape[0]
    fused = _fuse_table(embed_table, W, b)
    idx_flat = inputs.T.reshape(-1)
    out_flat = _sc_gather(fused, idx_flat)
    return out_flat.reshape(L, O, B).transpose(2, 0, 1)


# speedup vs baseline: 16.7747x; 1.0067x over previous
import functools

import jax
import jax.numpy as jnp
from jax import lax
from jax.experimental import pallas as pl
from jax.experimental.pallas import tpu as pltpu
from jax.experimental.pallas import tpu_sc as plsc

_NC = 2
_NS = 16
_NW = _NC * _NS
_D = 128
_K = 512
_O = 2
_PB = 16384


def _fuse_body(t_ref, w_ref, b_ref, o_ref):
    o_ref[...] = (
        jnp.dot(t_ref[...], w_ref[...], preferred_element_type=jnp.float32)
        + b_ref[...]
    )


def _fuse_table(table, W, b):
    V, E = table.shape
    O = W.shape[0]
    rows_per = 64
    w_in = rows_per * E
    w_out = rows_per * _D
    flat_rows = V // rows_per
    t_flat = table.reshape(flat_rows, w_in)
    w_pad = jnp.zeros((_D, E), jnp.float32).at[:O].set(W)
    b_pad = jnp.zeros((_D,), jnp.float32).at[:O].set(b)
    w_big = jnp.kron(jnp.eye(rows_per, dtype=jnp.float32), w_pad.T)
    b_tile = jnp.tile(b_pad, rows_per)[None, :]
    blk = 256
    grid = (flat_rows + blk - 1) // blk
    out_flat = pl.pallas_call(
        _fuse_body,
        grid=(grid,),
        in_specs=[
            pl.BlockSpec((blk, w_in), lambda i: (i, 0)),
            pl.BlockSpec((w_in, w_out), lambda i: (0, 0)),
            pl.BlockSpec((1, w_out), lambda i: (0, 0)),
        ],
        out_specs=pl.BlockSpec((blk, w_out), lambda i: (i, 0)),
        out_shape=jax.ShapeDtypeStruct((flat_rows, w_out), jnp.float32),
    )(t_flat, w_big, b_tile)
    return out_flat.reshape(V, _D)


def _sc_gather(fused, idx):
    n = idx.shape[0]
    per_w = n // _NW
    stages = per_w // _K
    mesh = plsc.VectorSubcoreMesh(
        core_axis_name="c", subcore_axis_name="s",
        num_cores=_NC, num_subcores=_NS,
    )

    @functools.partial(
        pl.kernel,
        mesh=mesh,
        compiler_params=pltpu.CompilerParams(needs_layout_passes=False),
        out_type=jax.ShapeDtypeStruct((n // _PB, _O * _PB), jnp.float32),
        scratch_types=[
            pltpu.VMEM((_K,), jnp.int32),
            pltpu.VMEM((_K, _D), jnp.float32),
            pltpu.VMEM((_K * _O,), jnp.float32),
            pltpu.SemaphoreType.DMA,
        ],
    )
    def gather_kernel(table_hbm, idx_hbm, out_hbm, idx_v, rows_v, com_v, sem):
        wid = lax.axis_index("s") * _NC + lax.axis_index("c")
        base = wid * per_w
        lane = lax.iota(jnp.int32, 16)
        live = lane < _O

        def stage(s, carry):
            p0 = base + s * _K
            h = p0 // _PB
            b0 = p0 - h * _PB
            pltpu.sync_copy(idx_hbm.at[pl.ds(p0, _K)], idx_v)
            pltpu.async_copy(table_hbm.at[idx_v], rows_v, sem).wait()

            def compact(g, c2):
                for u in range(8):
                    r = g * 8 + u
                    rhi = (r // 128) * (_O * 128)
                    rlo = r % 128
                    v = rows_v[r, pl.ds(0, 16)]
                    plsc.store_scatter(
                        com_v, [lane * 128 + (rhi + rlo)], v, mask=live
                    )
                return c2

            lax.fori_loop(0, _K // 8, compact, 0)
            pltpu.sync_copy(
                com_v, out_hbm.at[h, pl.ds(b0 * _O, _K * _O)]
            )
            return carry

        lax.fori_loop(0, stages, stage, 0)

    return gather_kernel(fused, idx)


def kernel(inputs, embed_table, W, b):
    B, L = inputs.shape
    O = W.shape[0]
    fused = _fuse_table(embed_table, W, b)
    idx_flat = inputs.T.reshape(-1)
    out2d = _sc_gather(fused, idx_flat)
    out4 = out2d.reshape(L, B // 128, O, 128)
    return out4.transpose(1, 3, 0, 2).reshape(B, L, O)
